# Initial kernel scaffold; baseline (speedup 1.0000x reference)
import jax
import jax.numpy as jnp
from jax.experimental import pallas as pl


def kernel(inputs, embeddings):
    raise NotImplementedError("write your pallas kernel here")


# speedup vs baseline: 4.9558x; 4.9558x over previous
import functools

import jax
import jax.numpy as jnp
from jax import lax
from jax.experimental import pallas as pl
from jax.experimental.pallas import tpu as pltpu
from jax.experimental.pallas import tpu_sc as plsc

MODEL_DIM = 128
SCALE = float(MODEL_DIM) ** 0.5

NUM_CORES = 2
NUM_SUBCORES = 16
LANES = 16
NUM_WORKERS = NUM_CORES * NUM_SUBCORES

CHUNK = 128
IDX_BLK = 8


@functools.partial(jax.jit, static_argnames=("n_rows",))
def _gather_scale(idx2d, table, n_rows):
  d = table.shape[1]
  n_chunks = idx2d.shape[0]
  ch_per_w = n_chunks // NUM_WORKERS
  blk_per_w = ch_per_w // IDX_BLK

  mesh = plsc.VectorSubcoreMesh(core_axis_name="c", subcore_axis_name="s")

  @functools.partial(
      pl.kernel,
      mesh=mesh,
      out_type=jax.ShapeDtypeStruct((n_rows, d), jnp.float32),
      scratch_types=[
          pltpu.VMEM((IDX_BLK, CHUNK), jnp.int32),
          pltpu.VMEM((CHUNK, d), jnp.float32),
          pltpu.SemaphoreType.DMA,
      ],
  )
  def k(table_hbm, idx_hbm, out_hbm, idx_v, rows_v, sem):
    wid = lax.axis_index("s") * NUM_CORES + lax.axis_index("c")
    ch_base = wid * ch_per_w

    def blk_body(bi, _):
      pltpu.sync_copy(idx_hbm.at[pl.ds(ch_base + bi * IDX_BLK, IDX_BLK)],
                      idx_v)

      def ch_body(j, _):
        pltpu.async_copy(table_hbm.at[idx_v.at[j]], rows_v, sem).wait()

        def row_body(r, _):
          for t in range(d // LANES):
            sl = pl.ds(t * LANES, LANES)
            rows_v[r, sl] = rows_v[r, sl] * SCALE
          return 0

        lax.fori_loop(0, CHUNK, row_body, 0, unroll=2)
        pltpu.sync_copy(
            rows_v, out_hbm.at[pl.ds((ch_base + bi * IDX_BLK + j) * CHUNK,
                                     CHUNK)])
        return 0

      lax.fori_loop(0, IDX_BLK, ch_body, 0)
      return 0

    lax.fori_loop(0, blk_per_w, blk_body, 0)

  return k(table, idx2d)


def kernel(inputs, embeddings):
  b, t = inputs.shape
  n_rows = b * t
  idx2d = inputs.reshape(n_rows // CHUNK, CHUNK).astype(jnp.int32)
  out = _gather_scale(idx2d, embeddings, n_rows)
  return out.reshape(b, t, embeddings.shape[1])


# speedup vs baseline: 8.6097x; 1.7373x over previous
import functools

import jax
import jax.numpy as jnp
from jax import lax
from jax.experimental import pallas as pl
from jax.experimental.pallas import tpu as pltpu
from jax.experimental.pallas import tpu_sc as plsc

MODEL_DIM = 128
SCALE = float(MODEL_DIM) ** 0.5

NUM_CORES = 2
NUM_SUBCORES = 16
LANES = 16
NUM_WORKERS = NUM_CORES * NUM_SUBCORES

CHUNK = 128
NBUF = 2


@functools.partial(jax.jit, static_argnames=("n_rows",))
def _gather_scale(idx2d, table, n_rows):
  d = table.shape[1]
  n_chunks = idx2d.shape[0]
  ch_per_w = n_chunks // NUM_WORKERS

  mesh = plsc.VectorSubcoreMesh(core_axis_name="c", subcore_axis_name="s")

  @functools.partial(
      pl.kernel,
      mesh=mesh,
      out_type=jax.ShapeDtypeStruct((n_rows, d), jnp.float32),
      scratch_types=[
          pltpu.VMEM((ch_per_w, CHUNK), jnp.int32),
          pltpu.VMEM((NBUF, CHUNK, d), jnp.float32),
          pltpu.SemaphoreType.DMA,
          pltpu.SemaphoreType.DMA,
          pltpu.SemaphoreType.DMA,
          pltpu.SemaphoreType.DMA,
      ],
  )
  def k(table_hbm, idx_hbm, out_hbm, idx_v, rows, g0, g1, s0, s1):
    wid = lax.axis_index("s") * NUM_CORES + lax.axis_index("c")
    ch_base = wid * ch_per_w
    gsems = [g0, g1]
    ssems = [s0, s1]
    bufs = [rows.at[b] for b in range(NBUF)]

    pltpu.sync_copy(idx_hbm.at[pl.ds(ch_base, ch_per_w)], idx_v)

    def gather(i, b):
      pltpu.async_copy(table_hbm.at[idx_v.at[i]], bufs[b], gsems[b])

    def wait_gather(b):
      pltpu.make_async_copy(table_hbm.at[idx_v.at[0]], bufs[b],
                            gsems[b]).wait()

    def store(i, b):
      pltpu.async_copy(bufs[b],
                       out_hbm.at[pl.ds((ch_base + i) * CHUNK, CHUNK)],
                       ssems[b])

    def wait_store(b):
      pltpu.make_async_copy(bufs[b], out_hbm.at[pl.ds(0, CHUNK)],
                            ssems[b]).wait()

    gather(0, 0)

    def pair_body(g, _):
      for b in range(NBUF):
        i = NBUF * g + b
        q = (b + 1) % NBUF

        @pl.when(i + 1 < ch_per_w)
        def _():
          @pl.when(i >= 1)
          def _():
            wait_store(q)
          gather(i + 1, q)

        wait_gather(b)

        def row_body(r, _):
          for t in range(d // LANES):
            sl = pl.ds(t * LANES, LANES)
            rows[b, r, sl] = rows[b, r, sl] * SCALE
          return 0

        lax.fori_loop(0, CHUNK, row_body, 0, unroll=2)
        store(i, b)
      return 0

    lax.fori_loop(0, ch_per_w // NBUF, pair_body, 0)
    for b in range(NBUF):
      wait_store(b)

  return k(table, idx2d)


def kernel(inputs, embeddings):
  b, t = inputs.shape
  n_rows = b * t
  idx2d = inputs.reshape(n_rows // CHUNK, CHUNK).astype(jnp.int32)
  out = _gather_scale(idx2d, embeddings, n_rows)
  return out.reshape(b, t, embeddings.shape[1])
